# baseline (speedup 1.0000x reference)
import functools

import jax
import jax.numpy as jnp
from jax import lax
from jax.experimental import pallas as pl
from jax.experimental.pallas import tpu as pltpu
from jax.experimental.pallas import tpu_sc as plsc

PANEL_W = 256
NBUF = 4
RING = 32
NB = 128
B_BITS = 14
L = 16


@functools.cache
def _make_gather(batch, vocab, dim):
    info = plsc.get_sparse_core_info()
    nc, ns = info.num_cores, info.num_subcores
    nw = nc * ns
    n_full = vocab // PANEL_W
    tail_w = vocab - n_full * PANEL_W
    per, rem = divmod(n_full, nw)
    assert per + 2 < NB
    assert batch <= (1 << B_BITS)
    assert (per + 1) * PANEL_W + tail_w < (1 << (31 - B_BITS))
    out_rows = batch + nw
    assert out_rows % 8 == 0 and batch % L == 0

    mesh = plsc.VectorSubcoreMesh(core_axis_name="c", subcore_axis_name="s")

    @functools.partial(
        pl.kernel,
        mesh=mesh,
        out_type=jax.ShapeDtypeStruct((out_rows, 2 * dim), jnp.float32),
        scratch_types=[
            pltpu.VMEM((batch,), jnp.int32),
            pltpu.VMEM((batch + L,), jnp.int32),
            pltpu.VMEM((batch + L,), jnp.int32),
            pltpu.VMEM((dim, PANEL_W), jnp.float32),
            pltpu.VMEM((dim, PANEL_W), jnp.float32),
            pltpu.VMEM((dim, PANEL_W), jnp.float32),
            pltpu.VMEM((dim, PANEL_W), jnp.float32),
            pltpu.VMEM((RING, 2 * dim), jnp.float32),
            pltpu.VMEM((1, RING), jnp.int32),
            pltpu.VMEM((NB,), jnp.int32),
            pltpu.VMEM((NB,), jnp.int32),
            pltpu.VMEM((NB,), jnp.int32),
            pltpu.SemaphoreType.DMA,
            pltpu.SemaphoreType.DMA,
            pltpu.SemaphoreType.DMA,
            pltpu.SemaphoreType.DMA,
        ],
        compiler_params=pltpu.CompilerParams(use_tc_tiling_on_sc=True,
                                             needs_layout_passes=False),
    )
    def k(idx_hbm, tt_hbm, tail_hbm, out_hbm,
          idx_v, bkt, srt, pan0, pan1, pan2, pan3,
          ring_v, rd_v, hist_v, starts_v, offs_v, sm0, sm1, sm2, sm3):
        wid = lax.axis_index("s") * nc + lax.axis_index("c")
        iota = lax.broadcasted_iota(jnp.int32, (L,), 0)
        zeros = jnp.zeros((L,), jnp.int32)
        ones = jnp.ones((L,), jnp.int32)
        dummy = jnp.full((L,), batch + wid, jnp.int32)
        lane0 = iota == 0
        p_shift = B_BITS + PANEL_W.bit_length() - 1

        n_my = per + jnp.where(wid < rem, 1, 0)
        p0 = wid * per + jnp.minimum(wid, rem)
        lo = p0 * PANEL_W
        hi = lo + n_my * PANEL_W
        hi = jnp.where(wid == nw - 1, vocab, hi)

        bufs = [pan0, pan1, pan2, pan3]
        sems = [sm0, sm1, sm2, sm3]

        def start_raw(p, buf, s):
            off = pl.multiple_of(p * PANEL_W + lo, PANEL_W)
            pltpu.async_copy(tt_hbm.at[:, pl.ds(off, PANEL_W)], buf, s)

        def wait_raw(buf, s):
            pltpu.make_async_copy(tt_hbm.at[:, pl.ds(0, PANEL_W)],
                                  buf, s).wait()

        for j in range(NBUF - 1):
            @pl.when(j < n_my)
            def _(j=j):
                start_raw(jnp.int32(j), bufs[j], sems[j])

        pltpu.sync_copy(idx_hbm, idx_v)

        def reset_rd():
            for g in range(RING // L):
                plsc.store_scatter(rd_v.at[...], [zeros, iota + g * L], dummy)

        reset_rd()
        for g in range(NB // L):
            hist_v[pl.ds(g * L, L)] = zeros

        def scan_body(kk, blen):
            iv = idx_v[pl.ds(kk * L, L)]
            m = (iv >= lo) & (iv < hi)
            v = ((iv - lo) << B_BITS) | (iota + kk * L)
            plsc.store_compressed(bkt.at[pl.ds(blen, L)], v, mask=m)
            return blen + plsc.all_reduce_population_count(m)[0]

        blen = lax.fori_loop(0, batch // L, scan_body, jnp.int32(0))

        def hist_body(kk, c):
            m = (iota + kk * L) < blen
            v = bkt[pl.ds(kk * L, L)]
            pv = jnp.where(m, v >> p_shift, NB - 1)
            plsc.addupdate_scatter(hist_v.at[...], [pv], ones, mask=m)
            return c

        lax.fori_loop(0, (blen + L - 1) // L, hist_body, jnp.int32(0))

        carry = jnp.int32(0)
        for g in range(NB // L):
            hv = hist_v[pl.ds(g * L, L)]
            s = plsc.cumsum(hv) + carry
            starts_v[pl.ds(g * L, L)] = s - hv
            offs_v[pl.ds(g * L, L)] = s - hv
            carry = s[L - 1]

        def place_body(t, c):
            tv = jnp.full((L,), t, jnp.int32)
            v = plsc.load_gather(bkt.at[...], [tv])
            pv = v >> p_shift
            dv = plsc.load_gather(offs_v.at[...], [pv])
            plsc.store_scatter(srt.at[...], [dv], v, mask=lane0)
            plsc.store_scatter(offs_v.at[...], [pv], dv + ones, mask=lane0)
            return c

        lax.fori_loop(0, blen, place_body, jnp.int32(0))

        def bin_bounds(p):
            pv = jnp.full((L,), p, jnp.int32)
            sp = plsc.load_gather(starts_v.at[...], [pv])[0]
            ep_v = plsc.load_gather(starts_v.at[...], [pv + ones])
            return sp, ep_v[0]

        def flush(rp):
            pltpu.sync_copy(ring_v, out_hbm.at[rd_v.at[0]])
            reset_rd()
            return rp

        def extract(panel_ref, sp, ep, rp):
            def group_body(gi, rp):
                t0 = sp + gi * L
                m = (t0 + iota) < ep
                v = srt[pl.ds(t0, L)]
                cvec = jnp.where(m, (v >> B_BITS) & (PANEL_W - 1), 0)
                bvec = jnp.where(m, v & ((1 << B_BITS) - 1), batch + wid)
                rpv = rp + iota
                for d in range(dim):
                    dv = jnp.full((L,), d, jnp.int32)
                    vals = plsc.load_gather(panel_ref.at[...], [dv, cvec])
                    plsc.store_scatter(ring_v.at[...], [rpv, dv], vals)
                plsc.store_scatter(rd_v.at[...], [zeros, rpv], bvec)
                rp = rp + L

                @pl.when(rp == RING)
                def _():
                    flush(rp)

                return jnp.where(rp == RING, 0, rp)

            ng = (ep - sp + L - 1) // L
            return lax.fori_loop(0, ng, group_body, rp)

        def group4_body(g, rp):
            for j in range(NBUF):
                p = g * NBUF + j

                @pl.when(p < n_my)
                def _(j=j):
                    wait_raw(bufs[j], sems[j])

                pn = p + NBUF - 1

                @pl.when(pn < n_my)
                def _(j=j, pn=pn):
                    start_raw(pn, bufs[(j + NBUF - 1) % NBUF],
                              sems[(j + NBUF - 1) % NBUF])

                sp, ep = bin_bounds(p)
                rp = extract(bufs[j], sp, jnp.where(p < n_my, ep, sp), rp)
            return rp

        rp = lax.fori_loop(0, (per + 1 + NBUF - 1) // NBUF, group4_body,
                           jnp.int32(0))

        if tail_w:
            @pl.when(wid == nw - 1)
            def _():
                pltpu.sync_copy(tail_hbm, pan0.at[:, pl.ds(0, 128)])

            sp, _unused = bin_bounds(n_my)
            rp = extract(pan0, sp, blen, rp)

        flush(rp)

    return k


def kernel(unique_ids, table):
    batch, = unique_ids.shape
    vocab, dim = table.shape
    tail_start = (vocab // PANEL_W) * PANEL_W
    idx = unique_ids.astype(jnp.int32)
    tt = table.T
    if tail_start < vocab:
        tail = jnp.pad(table[tail_start:].T,
                       ((0, 0), (0, 128 - (vocab - tail_start))))
    else:
        tail = jnp.zeros((dim, 128), table.dtype)
    out_wide = _make_gather(batch, vocab, dim)(idx, tt, tail)
    return out_wide[:batch, :dim]


# speedup vs baseline: 1.1042x; 1.1042x over previous
import functools

import jax
import jax.numpy as jnp
from jax import lax
from jax.experimental import pallas as pl
from jax.experimental.pallas import tpu as pltpu
from jax.experimental.pallas import tpu_sc as plsc

PANEL_W = 512
RING = 128
L = 16


@functools.cache
def _make_gather(batch, vocab, dim):
    info = plsc.get_sparse_core_info()
    nc, ns = info.num_cores, info.num_subcores
    nw = nc * ns
    n_full = vocab // PANEL_W
    tail_w = vocab - n_full * PANEL_W
    per, rem = divmod(n_full, nw)
    out_rows = batch + nw
    assert out_rows % 8 == 0 and batch % L == 0

    mesh = plsc.VectorSubcoreMesh(core_axis_name="c", subcore_axis_name="s")

    @functools.partial(
        pl.kernel,
        mesh=mesh,
        out_type=jax.ShapeDtypeStruct((out_rows, 2 * dim), jnp.float32),
        scratch_types=[
            pltpu.VMEM((batch,), jnp.int32),
            pltpu.VMEM((batch + L,), jnp.int32),
            pltpu.VMEM((batch + L,), jnp.int32),
            pltpu.VMEM((dim, PANEL_W), jnp.float32),
            pltpu.VMEM((dim, max(tail_w, 1)), jnp.float32),
            pltpu.VMEM((RING, 2 * dim), jnp.float32),
            pltpu.VMEM((1, RING), jnp.int32),
            pltpu.VMEM((L,), jnp.int32),
            pltpu.VMEM((L,), jnp.int32),
            pltpu.SemaphoreType.DMA,
        ],
        compiler_params=pltpu.CompilerParams(use_tc_tiling_on_sc=True,
                                             needs_layout_passes=False),
    )
    def k(idx_hbm, tt_hbm, tail_hbm, out_hbm,
          idx_v, bkt_i, bkt_b, panel_v, tail_v, ring_v, rd_v, st_c, st_b,
          sem):
        wid = lax.axis_index("s") * nc + lax.axis_index("c")
        iota = lax.broadcasted_iota(jnp.int32, (L,), 0)
        zeros = jnp.zeros((L,), jnp.int32)
        dummy = jnp.full((L,), batch + wid, jnp.int32)
        lane0 = iota == 0

        n_my = per + jnp.where(wid < rem, 1, 0)
        p0 = wid * per + jnp.minimum(wid, rem)
        lo = p0 * PANEL_W
        hi = lo + n_my * PANEL_W
        hi = jnp.where(wid == nw - 1, vocab, hi)

        pltpu.sync_copy(idx_hbm, idx_v)

        def reset_rd():
            for g in range(RING // L):
                plsc.store_scatter(rd_v.at[...], [zeros, iota + g * L], dummy)

        reset_rd()

        def scan_body(kk, blen):
            iv = idx_v[pl.ds(kk * L, L)]
            bv = iota + kk * L
            m = (iv >= lo) & (iv < hi)
            plsc.store_compressed(bkt_i.at[pl.ds(blen, L)], iv, mask=m)
            plsc.store_compressed(bkt_b.at[pl.ds(blen, L)], bv, mask=m)
            return blen + plsc.all_reduce_population_count(m)[0]

        blen = lax.fori_loop(0, batch // L, scan_body, jnp.int32(0))
        nk = (blen + L - 1) // L

        def flush(rp):
            pltpu.sync_copy(ring_v, out_hbm.at[rd_v.at[0]])
            reset_rd()
            return rp

        def extract(panel_ref, off, width, rp):
            def rescan_body(kk, rp):
                iv = bkt_i[pl.ds(kk * L, L)]
                bv = bkt_b[pl.ds(kk * L, L)]
                valid = (iota + kk * L) < blen
                m = valid & (iv >= off) & (iv < off + width)
                cnt = plsc.all_reduce_population_count(m)[0]
                plsc.store_compressed(st_c.at[...], iv - off, mask=m)
                plsc.store_compressed(st_b.at[...], bv, mask=m)

                def match_body(t, rp):
                    tv = jnp.full((L,), t, jnp.int32)
                    cvec = plsc.load_gather(st_c.at[...], [tv])
                    bvec = plsc.load_gather(st_b.at[...], [tv])
                    rpv = jnp.full((L,), rp, jnp.int32)
                    for g in range(dim // L):
                        dvec = iota + g * L
                        vals = plsc.load_gather(panel_ref.at[...], [dvec, cvec])
                        plsc.store_scatter(ring_v.at[...], [rpv, dvec], vals)
                    plsc.store_scatter(rd_v.at[...], [zeros, rpv], bvec,
                                       mask=lane0)
                    rp = rp + 1

                    @pl.when(rp == RING)
                    def _():
                        flush(rp)

                    return jnp.where(rp == RING, 0, rp)

                return lax.fori_loop(0, cnt, match_body, rp)

            return lax.fori_loop(0, nk, rescan_body, rp)

        def panel_body(p, rp):
            off = pl.multiple_of((p0 + p) * PANEL_W, PANEL_W)
            pltpu.sync_copy(tt_hbm.at[:, pl.ds(off, PANEL_W)], panel_v)
            return extract(panel_v, off, PANEL_W, rp)

        rp = lax.fori_loop(0, n_my, panel_body, jnp.int32(0))

        if tail_w:
            @pl.when(wid == nw - 1)
            def _():
                pltpu.sync_copy(tail_hbm, tail_v)

            eff_w = jnp.where(wid == nw - 1, tail_w, 0)
            rp = extract(tail_v, jnp.int32(n_full * PANEL_W), eff_w, rp)

        flush(rp)

    return k


def kernel(unique_ids, table):
    batch, = unique_ids.shape
    vocab, dim = table.shape
    tail_start = (vocab // PANEL_W) * PANEL_W
    idx = unique_ids.astype(jnp.int32)
    tt = table.T
    tail = table[tail_start:].T if tail_start < vocab else table[:1].T
    out_wide = _make_gather(batch, vocab, dim)(idx, tt, tail)
    return out_wide[:batch, :dim]
